# baseline (speedup 1.0000x reference)
import functools

import jax
import jax.numpy as jnp
from jax import lax
from jax.experimental import pallas as pl
from jax.experimental.pallas import tpu as pltpu
from jax.experimental.pallas import tpu_sc as plsc

ROWS = 16384
PAIRS = 128
LANES = 16
PENALTY = 10.0


def _body(rows_per_w, chunk, nc,
          comp_hbm, idx_i_hbm, idx_j_hbm, out_hbm,
          idx_v, a_v, b_v, acc_v):
    wid = lax.axis_index("s") * nc + lax.axis_index("c")
    row0 = wid * rows_per_w

    pltpu.sync_copy(idx_i_hbm.at[pl.ds(0, LANES)], idx_v)
    base_i = idx_v[...][0]
    pltpu.sync_copy(idx_j_hbm.at[pl.ds(0, LANES)], idx_v)
    base_j = idx_v[...][0]

    w_i = pl.multiple_of((base_i // 8) * 8, 8)
    w_j = pl.multiple_of((base_j // 8) * 8, 8)
    off_i = base_i - w_i
    off_j = base_j - w_j
    width = PAIRS + 8

    nsteps = rows_per_w // chunk

    def step(s, acc):
        r0 = row0 + s * chunk
        pltpu.sync_copy(comp_hbm.at[pl.ds(r0, chunk), pl.ds(w_i, width)], a_v)
        pltpu.sync_copy(comp_hbm.at[pl.ds(r0, chunk), pl.ds(w_j, width)], b_v)

        def row(r, acc):
            for c in range(PAIRS // LANES):
                acc = acc + (a_v[r, pl.ds(off_i + c * LANES, LANES)]
                             * b_v[r, pl.ds(off_j + c * LANES, LANES)])
            return acc

        return lax.fori_loop(0, chunk, row, acc)

    acc = lax.fori_loop(0, nsteps, step, jnp.zeros((LANES,), jnp.float32))
    acc_v[...] = acc * (PENALTY / ROWS)
    pltpu.sync_copy(acc_v, out_hbm.at[wid])


def kernel(compositions, pair_indices_i, pair_indices_j):
    info = plsc.get_sparse_core_info()
    nc, ns = info.num_cores, info.num_subcores
    nw = nc * ns
    rows_per_w = ROWS // nw
    chunk = min(rows_per_w, 256)

    mesh = plsc.VectorSubcoreMesh(core_axis_name="c", subcore_axis_name="s")
    run = pl.kernel(
        functools.partial(_body, rows_per_w, chunk, nc),
        out_type=jax.ShapeDtypeStruct((nw, LANES), jnp.float32),
        mesh=mesh,
        compiler_params=pltpu.CompilerParams(use_tc_tiling_on_sc=False),
        scratch_types=[
            pltpu.VMEM((LANES,), jnp.int32),
            pltpu.VMEM((chunk, PAIRS + 8), jnp.float32),
            pltpu.VMEM((chunk, PAIRS + 8), jnp.float32),
            pltpu.VMEM((LANES,), jnp.float32),
        ],
    )
    partials = run(compositions,
                   pair_indices_i.astype(jnp.int32),
                   pair_indices_j.astype(jnp.int32))
    return jnp.sum(partials)


# speedup vs baseline: 1.6206x; 1.6206x over previous
import functools

import jax
import jax.numpy as jnp
from jax import lax
from jax.experimental import pallas as pl
from jax.experimental.pallas import tpu as pltpu
from jax.experimental.pallas import tpu_sc as plsc

ROWS = 16384
PAIRS = 128
LANES = 16
PENALTY = 10.0


def _body(rows_per_w, chunk, nc,
          comp_hbm, idx_i_hbm, idx_j_hbm, out_hbm,
          idx_v, a_v, b_v, acc_v):
    wid = lax.axis_index("s") * nc + lax.axis_index("c")
    row0 = wid * rows_per_w

    pltpu.sync_copy(idx_i_hbm.at[pl.ds(0, LANES)], idx_v)
    base_i = idx_v[...][0]
    pltpu.sync_copy(idx_j_hbm.at[pl.ds(0, LANES)], idx_v)
    base_j = idx_v[...][0]

    w_i = pl.multiple_of((base_i // 128) * 128, 128)
    w_j = pl.multiple_of((base_j // 128) * 128, 128)
    off_i = base_i - w_i
    off_j = base_j - w_j
    width = 2 * PAIRS

    nsteps = rows_per_w // chunk

    iota = lax.iota(jnp.int32, LANES)
    ci = [off_i + c * LANES + iota for c in range(PAIRS // LANES)]
    cj = [off_j + c * LANES + iota for c in range(PAIRS // LANES)]

    def step(s, acc):
        r0 = row0 + s * chunk
        pltpu.sync_copy(comp_hbm.at[pl.ds(r0, chunk), pl.ds(w_i, width)], a_v)
        pltpu.sync_copy(comp_hbm.at[pl.ds(r0, chunk), pl.ds(w_j, width)], b_v)

        def row(r, acc):
            rv = jnp.full((LANES,), r, jnp.int32)
            for c in range(PAIRS // LANES):
                av = plsc.load_gather(a_v, [rv, ci[c]])
                bv = plsc.load_gather(b_v, [rv, cj[c]])
                acc = acc + av * bv
            return acc

        return lax.fori_loop(0, chunk, row, acc)

    acc = lax.fori_loop(0, nsteps, step, jnp.zeros((LANES,), jnp.float32))
    acc_v[...] = acc * (PENALTY / ROWS)
    pltpu.sync_copy(acc_v, out_hbm.at[wid])


def kernel(compositions, pair_indices_i, pair_indices_j):
    info = plsc.get_sparse_core_info()
    nc, ns = info.num_cores, info.num_subcores
    nw = nc * ns
    rows_per_w = ROWS // nw
    chunk = min(rows_per_w, 128)

    mesh = plsc.VectorSubcoreMesh(core_axis_name="c", subcore_axis_name="s")
    run = pl.kernel(
        functools.partial(_body, rows_per_w, chunk, nc),
        out_type=jax.ShapeDtypeStruct((nw, LANES), jnp.float32),
        mesh=mesh,
        compiler_params=pltpu.CompilerParams(needs_layout_passes=False),
        scratch_types=[
            pltpu.VMEM((LANES,), jnp.int32),
            pltpu.VMEM((chunk, 2 * PAIRS), jnp.float32),
            pltpu.VMEM((chunk, 2 * PAIRS), jnp.float32),
            pltpu.VMEM((LANES,), jnp.float32),
        ],
    )
    partials = run(compositions,
                   pair_indices_i.astype(jnp.int32),
                   pair_indices_j.astype(jnp.int32))
    return jnp.sum(partials)


# speedup vs baseline: 4.4524x; 2.7474x over previous
import functools

import jax
import jax.numpy as jnp
from jax import lax
from jax.experimental import pallas as pl
from jax.experimental.pallas import tpu as pltpu
from jax.experimental.pallas import tpu_sc as plsc

PAIRS = 128
LANES = 16
PENALTY = 10.0
CHUNK = 4096
UNROLL = 8


def _body(batch, pairs_per_w, nc,
          comp_hbm, idx_i_hbm, idx_j_hbm, out_hbm,
          ii_v, jj_v, iv_list, rows_v, acc_v, sem0, sem1):
    wid = lax.axis_index("s") * nc + lax.axis_index("c")
    p0 = wid * pairs_per_w

    pltpu.sync_copy(idx_i_hbm, ii_v)
    pltpu.sync_copy(idx_j_hbm, jj_v)

    iota = lax.iota(jnp.int32, LANES)
    first2 = iota < 2
    for k in range(pairs_per_w):
        pv = jnp.full((LANES,), p0 + k, jnp.int32)
        gi = plsc.load_gather(ii_v, [pv])
        gj = plsc.load_gather(jj_v, [pv])
        ivec = jnp.where(iota == 0, gi, gj)
        iv_list[k][...] = ivec

    nchunks = batch // CHUNK
    nsteps = pairs_per_w * nchunks
    sems = [sem0, sem1]

    def start(step, slot):
        k, c = step // nchunks, step % nchunks
        return pltpu.async_copy(
            comp_hbm.at[iv_list[k].at[pl.ds(0, 2)], pl.ds(c * CHUNK, CHUNK)],
            rows_v.at[slot], sems[slot])

    def fma_chunk(slot, accs):
        def it_body(it, accs):
            base = pl.multiple_of(it * (LANES * UNROLL), LANES * UNROLL)
            return tuple(
                accs[u] + (rows_v[slot, 0, pl.ds(base + u * LANES, LANES)]
                           * rows_v[slot, 1, pl.ds(base + u * LANES, LANES)])
                for u in range(UNROLL))
        return lax.fori_loop(0, CHUNK // (LANES * UNROLL), it_body, accs)

    accs = tuple(jnp.zeros((LANES,), jnp.float32) for _ in range(UNROLL))
    dmas = [start(0, 0), None]
    for s in range(nsteps):
        slot = s % 2
        if s + 1 < nsteps:
            dmas[1 - slot] = start(s + 1, 1 - slot)
        dmas[slot].wait()
        accs = fma_chunk(slot, accs)

    acc = accs[0]
    for u in range(1, UNROLL):
        acc = acc + accs[u]
    acc_v[...] = acc * (PENALTY / batch)
    pltpu.sync_copy(acc_v, out_hbm.at[wid])


def kernel(compositions, pair_indices_i, pair_indices_j):
    batch = compositions.shape[0]
    comp_t = compositions.T

    info = plsc.get_sparse_core_info()
    nc, ns = info.num_cores, info.num_subcores
    nw = nc * ns
    pairs_per_w = PAIRS // nw

    mesh = plsc.VectorSubcoreMesh(core_axis_name="c", subcore_axis_name="s")
    run = pl.kernel(
        functools.partial(_body, batch, pairs_per_w, nc),
        out_type=jax.ShapeDtypeStruct((nw, LANES), jnp.float32),
        mesh=mesh,
        compiler_params=pltpu.CompilerParams(needs_layout_passes=False),
        scratch_types=[
            pltpu.VMEM((PAIRS,), jnp.int32),
            pltpu.VMEM((PAIRS,), jnp.int32),
            [pltpu.VMEM((LANES,), jnp.int32) for _ in range(pairs_per_w)],
            pltpu.VMEM((2, 2, CHUNK), jnp.float32),
            pltpu.VMEM((LANES,), jnp.float32),
            pltpu.SemaphoreType.DMA,
            pltpu.SemaphoreType.DMA,
        ],
    )
    partials = run(comp_t,
                   pair_indices_i.astype(jnp.int32),
                   pair_indices_j.astype(jnp.int32))
    return jnp.sum(partials)
